# baseline (speedup 1.0000x reference)
import jax
import jax.numpy as jnp
from jax.experimental import pallas as pl
from jax.experimental.pallas import tpu as pltpu

NF = 512
ENC = 128
DEC = 128
ALPHA = 0.2
NEG = -1e30
HI = jax.lax.Precision.HIGHEST


def _leaky(x):
    return jnp.where(x >= 0, x, ALPHA * x)


def _elu(x):
    return jnp.where(x > 0, x, jnp.exp(jnp.minimum(x, 0.0)) - 1.0)


def _dot(a, b):
    return jnp.dot(a, b, preferred_element_type=jnp.float32, precision=HI)


def _masked_attention(E, mask_dst, mask_src, Wh):
    pair = jnp.logical_and(mask_dst, mask_src)
    logits = jnp.where(pair, E, NEG)
    m = jnp.max(logits, axis=1, keepdims=True)
    p = jnp.where(pair, jnp.exp(logits - m), 0.0)
    denom = jnp.sum(p, axis=1, keepdims=True) + 1e-16
    return _dot(p / denom, Wh)


def _sample(data_ref, dataT_ref, embed_ref, W_enc_ref, a_enc_ref, W_dec_ref,
            a_dec_ref, W_param_ref, b_param_ref, pred_ref, kld_ref):
    h0 = embed_ref[:NF, :]
    e_row = embed_ref[NF:NF + 1, :]

    W1 = W_enc_ref[0]
    W2 = W_enc_ref[1]
    a1s = jnp.reshape(a_enc_ref[0, :ENC], (ENC, 1))
    a1d = jnp.reshape(a_enc_ref[0, ENC:], (ENC, 1))
    a2s = jnp.reshape(a_enc_ref[1, :ENC], (ENC, 1))
    a2d = jnp.reshape(a_enc_ref[1, ENC:], (ENC, 1))
    Wd = W_dec_ref[...]
    ads = jnp.reshape(a_dec_ref[0, :DEC], (DEC, 1))
    add = jnp.reshape(a_dec_ref[0, DEC:], (DEC, 1))

    mask_r = data_ref[0] != 0
    mask_c = dataT_ref[0] != 0

    Wh1 = _dot(h0, W1)
    es1 = _dot(Wh1, a1s)
    ed1 = _dot(Wh1, a1d)
    E1 = _leaky(ed1 + es1.T)
    h1 = _elu(_masked_attention(E1, mask_c, mask_r, Wh1))

    Wh2 = _dot(h1, W2)
    es2 = _dot(Wh2, a2s)
    ed2 = _dot(Wh2, a2d)
    E2 = _leaky(ed2 + es2.T)
    h2 = _elu(_masked_attention(E2, mask_c, mask_r, Wh2))

    par = _dot(h2, W_param_ref[...]) + b_param_ref[...]
    mean = par[:, :DEC]
    sigma = par[:, DEC:]

    maskf = mask_c.astype(jnp.float32)
    cnt = jnp.sum(maskf)
    term = jnp.exp(sigma) - sigma - 1.0 + mean * mean
    kld_ref[...] = jnp.reshape(0.5 * jnp.sum(term * maskf) / cnt, (1, 1, 1))

    wh_last = _dot(e_row, Wd)
    es_last = _dot(wh_last, ads)
    ed_last = _dot(wh_last, add)
    logit_last = _leaky(ed_last + es_last)
    esd = _dot(mean, _dot(Wd, ads))
    lg = _leaky(ed_last + esd.T)
    lg = jnp.where(mask_r, lg, NEG)
    m = jnp.maximum(jnp.max(lg, axis=1, keepdims=True), logit_last)
    p = jnp.where(mask_r, jnp.exp(lg - m), 0.0)
    p_last = jnp.exp(logit_last - m)
    denom = jnp.sum(p, axis=1, keepdims=True) + p_last + 1e-16
    dec = (_dot(_dot(p, mean), Wd) + p_last * wh_last) / denom
    pred_ref[0] = jax.nn.relu(dec)


def _combine(preds_ref, klds_ref, W_out1_ref, b_out1_ref, W_out2_ref,
             b_out2_ref, pred_ref, kld_ref):
    hidden = jax.nn.relu(_dot(preds_ref[...], W_out1_ref[...]) + b_out1_ref[...])
    pred_ref[...] = _dot(hidden, W_out2_ref[...]) + b_out2_ref[...]
    kld_ref[...] = jnp.sum(klds_ref[...], axis=1, keepdims=True)


def _const_spec(ndim):
    return pl.BlockSpec(None, lambda i: (0,) * ndim)


def kernel(data, embed, W_enc, a_enc, W_dec, a_dec, W_param, b_param,
           W_out1, b_out1, W_out2, b_out2):
    data = data.astype(jnp.int32)
    preds, klds = pl.pallas_call(
        _sample,
        grid=(8,),
        in_specs=[
            pl.BlockSpec((1, 1, NF), lambda i: (i, 0, 0)),
            pl.BlockSpec((1, NF, 1), lambda i: (i, 0, 0)),
            _const_spec(2),
            _const_spec(3),
            _const_spec(2),
            _const_spec(2),
            _const_spec(2),
            _const_spec(2),
            _const_spec(2),
        ],
        out_specs=(
            pl.BlockSpec((1, 1, DEC), lambda i: (i, 0, 0)),
            pl.BlockSpec((1, 1, 1), lambda i: (i, 0, 0)),
        ),
        out_shape=(
            jax.ShapeDtypeStruct((8, 1, DEC), jnp.float32),
            jax.ShapeDtypeStruct((8, 1, 1), jnp.float32),
        ),
        compiler_params=pltpu.CompilerParams(
            dimension_semantics=("parallel",)),
    )(
        data.reshape(8, 1, NF),
        data.reshape(8, NF, 1),
        embed,
        W_enc.reshape(2, ENC, ENC),
        a_enc.reshape(2, 2 * ENC),
        W_dec.reshape(ENC, DEC),
        a_dec.reshape(1, 2 * DEC),
        W_param,
        b_param.reshape(1, 2 * ENC),
    )
    prediction, kld = pl.pallas_call(
        _combine,
        out_shape=(
            jax.ShapeDtypeStruct((8, 1), jnp.float32),
            jax.ShapeDtypeStruct((1, 1), jnp.float32),
        ),
    )(
        preds.reshape(8, DEC),
        klds.reshape(1, 8),
        W_out1,
        b_out1.reshape(1, DEC),
        W_out2,
        b_out2.reshape(1, 1),
    )
    return prediction, kld[0, 0]


# speedup vs baseline: 1.2849x; 1.2849x over previous
import jax
import jax.numpy as jnp
from jax.experimental import pallas as pl

NF = 512
ENC = 128
DEC = 128
ALPHA = 0.2
NEG = -1e30
HI = jax.lax.Precision.HIGHEST


def _leaky(x):
    return jnp.where(x >= 0, x, ALPHA * x)


def _elu(x):
    return jnp.where(x > 0, x, jnp.exp(jnp.minimum(x, 0.0)) - 1.0)


def _dot(a, b):
    return jnp.dot(a, b, preferred_element_type=jnp.float32, precision=HI)


def _masked_attention(E, mask_dst, mask_src, Wh):
    pair = jnp.logical_and(mask_dst, mask_src)
    logits = jnp.where(pair, E, NEG)
    m = jnp.max(logits, axis=1, keepdims=True)
    p = jnp.where(pair, jnp.exp(logits - m), 0.0)
    denom = jnp.sum(p, axis=1, keepdims=True) + 1e-16
    return _dot(p / denom, Wh)


def _fwd(data_ref, dataT_ref, embed_ref, W_enc_ref, a_enc_ref, W_dec_ref,
         a_dec_ref, W_param_ref, b_param_ref, W_out1_ref, b_out1_ref,
         W_out2_ref, b_out2_ref, pred_ref, kld_ref):
    h0 = embed_ref[:NF, :]
    e_row = embed_ref[NF:NF + 1, :]

    W1 = W_enc_ref[0]
    W2 = W_enc_ref[1]
    a1s = jnp.reshape(a_enc_ref[0, :ENC], (ENC, 1))
    a1d = jnp.reshape(a_enc_ref[0, ENC:], (ENC, 1))
    a2s = jnp.reshape(a_enc_ref[1, :ENC], (ENC, 1))
    a2d = jnp.reshape(a_enc_ref[1, ENC:], (ENC, 1))
    Wd = W_dec_ref[...]
    ads = jnp.reshape(a_dec_ref[0, :DEC], (DEC, 1))
    add = jnp.reshape(a_dec_ref[0, DEC:], (DEC, 1))
    W_param = W_param_ref[...]
    b_param = b_param_ref[...]

    Wh1 = _dot(h0, W1)
    es1 = _dot(Wh1, a1s)
    ed1 = _dot(Wh1, a1d)
    E1 = _leaky(ed1 + es1.T)

    wh_last = _dot(e_row, Wd)
    es_last = _dot(wh_last, ads)
    ed_last = _dot(wh_last, add)
    logit_last = _leaky(ed_last + es_last)
    wd_ads = _dot(Wd, ads)

    preds = []
    klds = []
    for i in range(8):
        mask_c = dataT_ref[:, i:i + 1] != 0
        mask_r = data_ref[i:i + 1, :] != 0

        h1 = _elu(_masked_attention(E1, mask_c, mask_r, Wh1))

        Wh2 = _dot(h1, W2)
        es2 = _dot(Wh2, a2s)
        ed2 = _dot(Wh2, a2d)
        E2 = _leaky(ed2 + es2.T)
        h2 = _elu(_masked_attention(E2, mask_c, mask_r, Wh2))

        par = _dot(h2, W_param) + b_param
        mean = par[:, :DEC]
        sigma = par[:, DEC:]

        maskf = mask_c.astype(jnp.float32)
        cnt = jnp.sum(maskf)
        term = jnp.exp(sigma) - sigma - 1.0 + mean * mean
        klds.append(jnp.reshape(0.5 * jnp.sum(term * maskf) / cnt, (1, 1)))

        esd = _dot(mean, wd_ads)
        lg = _leaky(ed_last + esd.T)
        lg = jnp.where(mask_r, lg, NEG)
        m = jnp.maximum(jnp.max(lg, axis=1, keepdims=True), logit_last)
        p = jnp.where(mask_r, jnp.exp(lg - m), 0.0)
        p_last = jnp.exp(logit_last - m)
        denom = jnp.sum(p, axis=1, keepdims=True) + p_last + 1e-16
        dec = (_dot(_dot(p, mean), Wd) + p_last * wh_last) / denom
        preds.append(jax.nn.relu(dec))

    stacked = jnp.concatenate(preds, axis=0)
    hidden = jax.nn.relu(_dot(stacked, W_out1_ref[...]) + b_out1_ref[...])
    pred_ref[...] = _dot(hidden, W_out2_ref[...]) + b_out2_ref[...]
    kld_ref[...] = sum(klds[1:], klds[0])


def kernel(data, embed, W_enc, a_enc, W_dec, a_dec, W_param, b_param,
           W_out1, b_out1, W_out2, b_out2):
    data = data.astype(jnp.int32)
    out = pl.pallas_call(
        _fwd,
        out_shape=(
            jax.ShapeDtypeStruct((8, 1), jnp.float32),
            jax.ShapeDtypeStruct((1, 1), jnp.float32),
        ),
    )(
        data,
        data.T,
        embed,
        W_enc.reshape(2, ENC, ENC),
        a_enc.reshape(2, 2 * ENC),
        W_dec.reshape(ENC, DEC),
        a_dec.reshape(1, 2 * DEC),
        W_param,
        b_param.reshape(1, 2 * ENC),
        W_out1,
        b_out1.reshape(1, DEC),
        W_out2,
        b_out2.reshape(1, 1),
    )
    prediction, kld = out
    return prediction, kld[0, 0]
